# Initial kernel scaffold; baseline (speedup 1.0000x reference)
import jax
import jax.numpy as jnp
from jax import lax
from jax.experimental import pallas as pl
from jax.experimental.pallas import tpu as pltpu
from jax.experimental.pallas import tpu_sc as plsc

N_NODES = 50000
HALF_D = 32
E = 800000
N_LAYERS = 3
NC, NS = 2, 16

E_TILE = 50176
E_PAD = E_TILE * NS
IDX_ROWS = E_PAD // 128
T_STREAMS = E_TILE // 128
CH = 4
CHUNK_E = CH * 128
N_CHUNKS = T_STREAMS // CH
ROWS_PER_TILE = N_NODES // NS
Z_ROWS = 625
F_ROWS = 125


def _body(ego0, rows2, cols2, vals, out, t1, t2, t3,
          acc, gbuf, colb, rowb, valb, zbuf, fbuf, obuf, sem_g, sem_s):
    c = lax.axis_index("c")
    s = lax.axis_index("s")
    half_base = c * N_NODES + s * ROWS_PER_TILE

    zeros16 = jnp.zeros((16,), jnp.float32)

    @pl.loop(0, Z_ROWS)
    def _zb(r):
        zbuf[r, 0:16] = zeros16
        zbuf[r, 16:32] = zeros16

    def run_layer(src_tbl, dst_tbl):
        for k in range(ROWS_PER_TILE // Z_ROWS):
            pltpu.sync_copy(
                zbuf, acc.at[pl.ds(s * ROWS_PER_TILE + k * Z_ROWS, Z_ROWS)])
        plsc.subcore_barrier()

        @pl.loop(0, N_CHUNKS)
        def _chunk(ch):
            irow = s * T_STREAMS + ch * CH
            pltpu.sync_copy(cols2.at[pl.ds(c * IDX_ROWS + irow, CH)], colb)
            pltpu.sync_copy(rows2.at[pl.ds(irow, CH)], rowb)
            pltpu.sync_copy(
                vals.at[pl.ds(s * E_TILE + ch * CHUNK_E, CHUNK_E)], valb)
            gds = [
                pltpu.async_copy(src_tbl.at[colb.at[j]],
                                 gbuf.at[pl.ds(j * 128, 128)], sem_g)
                for j in range(CH)
            ]
            for d in gds:
                d.wait()

            @pl.loop(0, CHUNK_E, unroll=8)
            def _scale(e):
                v = valb[e]
                gbuf[e, 0:16] = gbuf[e, 0:16] * v
                gbuf[e, 16:32] = gbuf[e, 16:32] * v

            sds = [
                pltpu.async_copy(gbuf.at[pl.ds(j * 128, 128)],
                                 acc.at[rowb.at[j]], sem_s, add=True)
                for j in range(CH)
            ]
            for d in sds:
                d.wait()

        plsc.subcore_barrier()
        pltpu.sync_copy(acc.at[pl.ds(s * ROWS_PER_TILE, ROWS_PER_TILE)],
                        dst_tbl.at[pl.ds(half_base, ROWS_PER_TILE)])

    srcs = [ego0, t1, t2]
    dsts = [t1, t2, t3]
    for l in range(N_LAYERS):
        run_layer(srcs[l], dsts[l])

    @pl.loop(0, ROWS_PER_TILE // F_ROWS)
    def _fin(t):
        base = half_base + t * F_ROWS
        pltpu.sync_copy(ego0.at[pl.ds(base, F_ROWS)], fbuf.at[0])
        pltpu.sync_copy(t1.at[pl.ds(base, F_ROWS)], fbuf.at[1])
        pltpu.sync_copy(t2.at[pl.ds(base, F_ROWS)], fbuf.at[2])
        pltpu.sync_copy(t3.at[pl.ds(base, F_ROWS)], fbuf.at[3])

        @pl.loop(0, F_ROWS)
        def _avg(r):
            for h in (0, 16):
                obuf[r, h:h + 16] = (
                    fbuf[0, r, h:h + 16] + fbuf[1, r, h:h + 16]
                    + fbuf[2, r, h:h + 16] + fbuf[3, r, h:h + 16]) * 0.25

        pltpu.sync_copy(obuf, out.at[pl.ds(base, F_ROWS)])


_mesh = plsc.VectorSubcoreMesh(
    core_axis_name="c", subcore_axis_name="s", num_cores=NC, num_subcores=NS)

_tbl = jax.ShapeDtypeStruct((2 * N_NODES, HALF_D), jnp.float32)

_gcl = pl.kernel(
    _body,
    out_type=(_tbl, _tbl, _tbl, _tbl),
    mesh=_mesh,
    scratch_types=[
        pltpu.VMEM_SHARED((N_NODES, HALF_D), jnp.float32),
        pltpu.VMEM((CHUNK_E, HALF_D), jnp.float32),
        pltpu.VMEM((CH, 128), jnp.int32),
        pltpu.VMEM((CH, 128), jnp.int32),
        pltpu.VMEM((CHUNK_E,), jnp.float32),
        pltpu.VMEM((Z_ROWS, HALF_D), jnp.float32),
        pltpu.VMEM((4, F_ROWS, HALF_D), jnp.float32),
        pltpu.VMEM((F_ROWS, HALF_D), jnp.float32),
        pltpu.SemaphoreType.DMA,
        pltpu.SemaphoreType.DMA,
    ],
)


@jax.jit
def kernel(user_emb, item_emb, adj_rows, adj_cols, adj_vals):
    ego = jnp.concatenate([user_emb, item_emb], axis=0)
    ego_h = jnp.concatenate([ego[:, :HALF_D], ego[:, HALF_D:]], axis=0)
    pad = E_PAD - E
    rows_p = jnp.concatenate(
        [adj_rows.astype(jnp.int32), jnp.zeros((pad,), jnp.int32)])
    cols_p = jnp.concatenate(
        [adj_cols.astype(jnp.int32), jnp.zeros((pad,), jnp.int32)])
    vals_p = jnp.concatenate([adj_vals, jnp.zeros((pad,), jnp.float32)])
    cols2 = jnp.concatenate(
        [cols_p, cols_p + N_NODES]).reshape(2 * IDX_ROWS, 128)
    rows2 = rows_p.reshape(IDX_ROWS, 128)

    out, _, _, _ = _gcl(ego_h, rows2, cols2, vals_p)
    full = jnp.concatenate([out[:N_NODES], out[N_NODES:]], axis=1)
    return full[: N_NODES // 2], full[N_NODES // 2:]


# speedup vs baseline: 4.2377x; 4.2377x over previous
import jax
import jax.numpy as jnp
from jax import lax
from jax.experimental import pallas as pl
from jax.experimental.pallas import tpu as pltpu
from jax.experimental.pallas import tpu_sc as plsc

N_NODES = 50000
N_TBL = 50048
HALF_D = 32
E = 800000
N_LAYERS = 3
NC, NS = 2, 16

T_STREAMS = 396
E_TILE = T_STREAMS * 128
E_PAD = E_TILE * NS
IDX_ROWS = E_PAD // 128
ROWS_PER_TILE = N_TBL // NS
F_ROWS = 184


def _body(ego0, edata, out, t1, t2, t3,
          acc, ebuf, gbuf, fbuf, obuf, sem_g, sem_s):
    c = lax.axis_index("c")
    s = lax.axis_index("s")
    half_base = c * N_TBL + s * ROWS_PER_TILE
    ebase = c * IDX_ROWS + s * T_STREAMS
    zeros16 = jnp.zeros((16,), jnp.float32)

    def run_layer(src_tbl, dst_tbl):
        @pl.loop(0, 128)
        def _zb(r):
            gbuf[0, r, 0:16] = zeros16
            gbuf[0, r, 16:32] = zeros16

        for j in range(24):
            pltpu.sync_copy(
                gbuf.at[0],
                acc.at[pl.ds(s * ROWS_PER_TILE + j * 128, 128)])
        pltpu.sync_copy(
            gbuf.at[0].at[pl.ds(0, 56)],
            acc.at[pl.ds(s * ROWS_PER_TILE + 3072, 56)])
        plsc.subcore_barrier()

        @pl.loop(0, T_STREAMS)
        def _chunk(k):
            pltpu.sync_copy(edata.at[ebase + k], ebuf.at[0])
            pltpu.async_copy(src_tbl.at[ebuf.at[0].at[0]],
                             gbuf.at[0], sem_g).wait()

            @pl.loop(0, 8)
            def _scale(i):
                vv = plsc.bitcast(ebuf[0, 2, pl.ds(i * 16, 16)], jnp.float32)
                for q in range(16):
                    e = i * 16 + q
                    v = vv[q]
                    gbuf[0, e, 0:16] = gbuf[0, e, 0:16] * v
                    gbuf[0, e, 16:32] = gbuf[0, e, 16:32] * v

            pltpu.async_copy(gbuf.at[0], acc.at[ebuf.at[0].at[1]],
                             sem_s, add=True).wait()

        plsc.subcore_barrier()
        pltpu.sync_copy(acc.at[pl.ds(s * ROWS_PER_TILE, ROWS_PER_TILE)],
                        dst_tbl.at[pl.ds(half_base, ROWS_PER_TILE)])

    srcs = [ego0, t1, t2]
    dsts = [t1, t2, t3]
    for l in range(N_LAYERS):
        run_layer(srcs[l], dsts[l])

    @pl.loop(0, ROWS_PER_TILE // F_ROWS)
    def _fin(t):
        base = half_base + t * F_ROWS
        pltpu.sync_copy(ego0.at[pl.ds(base, F_ROWS)], obuf)
        for tbl in (t1, t2, t3):
            pltpu.sync_copy(tbl.at[pl.ds(base, F_ROWS)], fbuf)

            @pl.loop(0, F_ROWS)
            def _acc(r):
                obuf[r, 0:16] = obuf[r, 0:16] + fbuf[r, 0:16]
                obuf[r, 16:32] = obuf[r, 16:32] + fbuf[r, 16:32]

        @pl.loop(0, F_ROWS)
        def _avg(r):
            obuf[r, 0:16] = obuf[r, 0:16] * 0.25
            obuf[r, 16:32] = obuf[r, 16:32] * 0.25

        pltpu.sync_copy(obuf, out.at[pl.ds(base, F_ROWS)])


_mesh = plsc.VectorSubcoreMesh(
    core_axis_name="c", subcore_axis_name="s", num_cores=NC, num_subcores=NS)

_tbl = jax.ShapeDtypeStruct((2 * N_TBL, HALF_D), jnp.float32)

_gcl = pl.kernel(
    _body,
    out_type=(_tbl, _tbl, _tbl, _tbl),
    mesh=_mesh,
    compiler_params=pltpu.CompilerParams(
        use_tc_tiling_on_sc=False, needs_layout_passes=False),
    scratch_types=[
        pltpu.VMEM_SHARED((N_TBL, HALF_D), jnp.float32),
        pltpu.VMEM((2, 3, 128), jnp.int32),
        pltpu.VMEM((2, 128, HALF_D), jnp.float32),
        pltpu.VMEM((F_ROWS, HALF_D), jnp.float32),
        pltpu.VMEM((F_ROWS, HALF_D), jnp.float32),
        pltpu.SemaphoreType.DMA,
        pltpu.SemaphoreType.DMA,
    ],
)


@jax.jit
def kernel(user_emb, item_emb, adj_rows, adj_cols, adj_vals):
    ego = jnp.concatenate([user_emb, item_emb], axis=0)
    zrows = jnp.zeros((N_TBL - N_NODES, HALF_D), jnp.float32)
    ego_h = jnp.concatenate(
        [ego[:, :HALF_D], zrows, ego[:, HALF_D:], zrows], axis=0)
    pad = E_PAD - E
    rows_p = jnp.concatenate(
        [adj_rows.astype(jnp.int32), jnp.zeros((pad,), jnp.int32)])
    cols_p = jnp.concatenate(
        [adj_cols.astype(jnp.int32), jnp.zeros((pad,), jnp.int32)])
    vals_p = jnp.concatenate([adj_vals, jnp.zeros((pad,), jnp.float32)])
    cols3 = jnp.stack([cols_p, cols_p + N_TBL]).reshape(2, IDX_ROWS, 128)
    rows3 = jnp.broadcast_to(
        rows_p.reshape(1, IDX_ROWS, 128), (2, IDX_ROWS, 128))
    vals3 = jnp.broadcast_to(
        lax.bitcast_convert_type(vals_p, jnp.int32).reshape(1, IDX_ROWS, 128),
        (2, IDX_ROWS, 128))
    edata = jnp.stack([cols3, rows3, vals3], axis=2).reshape(
        2 * IDX_ROWS, 3, 128)

    out, _, _, _ = _gcl(ego_h, edata)
    full = jnp.concatenate(
        [out[:N_NODES], out[N_TBL:N_TBL + N_NODES]], axis=1)
    return full[: N_NODES // 2], full[N_NODES // 2:]


# speedup vs baseline: 10.0223x; 2.3650x over previous
import jax
import jax.numpy as jnp
from jax import lax
from jax.experimental import pallas as pl
from jax.experimental.pallas import tpu as pltpu
from jax.experimental.pallas import tpu_sc as plsc

N_NODES = 50000
N_TBL = 50048
HALF_D = 32
E = 800000
N_LAYERS = 3
NC, NS = 2, 16
NBUF = 6

T_STREAMS = 396
E_TILE = T_STREAMS * 128
E_PAD = E_TILE * NS
IDX_ROWS = E_PAD // 128
ROWS_PER_TILE = N_TBL // NS
N_GRP = T_STREAMS // NBUF


def _body(ego0, cols, rows, vals, out, t1, t2, t3,
          acc, colb, rowb, valb, gbuf, sem_e, sem_g, sem_s, sem_f):
    c = lax.axis_index("c")
    s = lax.axis_index("s")
    half_base = c * N_TBL + s * ROWS_PER_TILE
    ebase = s * T_STREAMS
    zeros16 = jnp.zeros((16,), jnp.float32)

    def run_layer(src_tbl, dst_tbl):
        @pl.loop(0, 128)
        def _zb(r):
            gbuf[0, r, 0:16] = zeros16
            gbuf[0, r, 16:32] = zeros16

        for q in range(24):
            pltpu.sync_copy(
                gbuf.at[0],
                acc.at[pl.ds(s * ROWS_PER_TILE + q * 128, 128)])
        pltpu.sync_copy(
            gbuf.at[0].at[pl.ds(0, 56)],
            acc.at[pl.ds(s * ROWS_PER_TILE + 3072, 56)])
        plsc.subcore_barrier()

        def stage(kk, j):
            pltpu.async_copy(cols.at[ebase + kk], colb.at[j], sem_e.at[j])
            pltpu.async_copy(rows.at[ebase + kk], rowb.at[j], sem_e.at[j])
            pltpu.async_copy(vals.at[ebase + kk], valb.at[j], sem_e.at[j])

        def wait_e(j):
            pltpu.make_async_copy(cols.at[ebase], colb.at[j],
                                  sem_e.at[j]).wait()
            pltpu.make_async_copy(rows.at[ebase], rowb.at[j],
                                  sem_e.at[j]).wait()
            pltpu.make_async_copy(vals.at[ebase], valb.at[j],
                                  sem_e.at[j]).wait()
            for i in range(8):
                sl = pl.ds(i * 16, 16)
                colb[j, sl] = colb[j, sl] + c * N_TBL

        def gather(j):
            pltpu.async_copy(src_tbl.at[colb.at[j]],
                             gbuf.at[j], sem_g.at[j])

        def wait_g(j):
            pltpu.make_async_copy(src_tbl.at[colb.at[j]],
                                  gbuf.at[j], sem_g.at[j]).wait()

        def scatter(j):
            pltpu.async_copy(gbuf.at[j], acc.at[rowb.at[j]],
                             sem_s.at[j], add=True)

        def wait_s(j):
            pltpu.make_async_copy(gbuf.at[j], acc.at[rowb.at[j]],
                                  sem_s.at[j]).wait()

        def scale(j):
            @pl.loop(0, 8)
            def _scale(i):
                vv = valb[j, pl.ds(i * 16, 16)]
                for q in range(16):
                    e = i * 16 + q
                    v = vv[q]
                    gbuf[j, e, 0:16] = gbuf[j, e, 0:16] * v
                    gbuf[j, e, 16:32] = gbuf[j, e, 16:32] * v

        def step(kk, j, ws=True, st=True, ga=True):
            wait_g(j)
            scale(j)
            scatter(j)
            if ws:
                wait_s((j + 4) % NBUF)
            if st:
                stage(kk + 4, (j + 4) % NBUF)
            if ga:
                wait_e((j + 3) % NBUF)
                gather((j + 3) % NBUF)

        for j in range(4):
            stage(j, j)
        for j in range(3):
            wait_e(j)
            gather(j)
        for k in range(NBUF):
            step(k, k, ws=(k >= 2))

        @pl.loop(1, N_GRP - 1)
        def _grp(g):
            for j in range(NBUF):
                step(NBUF * g + j, j)

        for k in range(T_STREAMS - NBUF, T_STREAMS):
            step(k, k % NBUF,
                 st=(k + 4 <= T_STREAMS - 1), ga=(k + 3 <= T_STREAMS - 1))
        for j in (4, 5):
            wait_s(j)

        plsc.subcore_barrier()
        pltpu.sync_copy(acc.at[pl.ds(s * ROWS_PER_TILE, ROWS_PER_TILE)],
                        dst_tbl.at[pl.ds(half_base, ROWS_PER_TILE)])

    srcs = [ego0, t1, t2]
    dsts = [t1, t2, t3]
    for l in range(N_LAYERS):
        run_layer(srcs[l], dsts[l])

    def fin_chunk(base, nrows):
        tbls = (ego0, t1, t2, t3)
        for i, tbl in enumerate(tbls):
            pltpu.async_copy(tbl.at[pl.ds(base, nrows)],
                             gbuf.at[i].at[pl.ds(0, nrows)], sem_f.at[i])
        for i, tbl in enumerate(tbls):
            pltpu.make_async_copy(tbl.at[pl.ds(base, nrows)],
                                  gbuf.at[i].at[pl.ds(0, nrows)],
                                  sem_f.at[i]).wait()

        @pl.loop(0, nrows)
        def _avg(r):
            for h in (0, 16):
                gbuf[4, r, h:h + 16] = (
                    (gbuf[0, r, h:h + 16] + gbuf[1, r, h:h + 16])
                    + (gbuf[2, r, h:h + 16] + gbuf[3, r, h:h + 16])) * 0.25

        pltpu.sync_copy(
            gbuf.at[4].at[pl.ds(0, nrows)],
            out.at[pl.ds(base - c * N_TBL, nrows),
                   pl.ds(c * HALF_D, HALF_D)])

    @pl.loop(0, 24)
    def _fin(t):
        fin_chunk(half_base + t * 128, 128)

    fin_chunk(half_base + 3072, 56)


_mesh = plsc.VectorSubcoreMesh(
    core_axis_name="c", subcore_axis_name="s", num_cores=NC, num_subcores=NS)

_tbl = jax.ShapeDtypeStruct((2 * N_TBL, HALF_D), jnp.float32)
_out_t = jax.ShapeDtypeStruct((N_TBL, 2 * HALF_D), jnp.float32)

_gcl = pl.kernel(
    _body,
    out_type=(_out_t, _tbl, _tbl, _tbl),
    mesh=_mesh,
    compiler_params=pltpu.CompilerParams(
        use_tc_tiling_on_sc=False, needs_layout_passes=False),
    scratch_types=[
        pltpu.VMEM_SHARED((N_TBL, HALF_D), jnp.float32),
        pltpu.VMEM((NBUF, 128), jnp.int32),
        pltpu.VMEM((NBUF, 128), jnp.int32),
        pltpu.VMEM((NBUF, 128), jnp.float32),
        pltpu.VMEM((NBUF, 128, HALF_D), jnp.float32),
        pltpu.SemaphoreType.DMA((NBUF,)),
        pltpu.SemaphoreType.DMA((NBUF,)),
        pltpu.SemaphoreType.DMA((NBUF,)),
        pltpu.SemaphoreType.DMA((4,)),
    ],
)


@jax.jit
def kernel(user_emb, item_emb, adj_rows, adj_cols, adj_vals):
    ego = jnp.concatenate([user_emb, item_emb], axis=0)
    zrows = jnp.zeros((N_TBL - N_NODES, HALF_D), jnp.float32)
    ego_h = jnp.concatenate(
        [ego[:, :HALF_D], zrows, ego[:, HALF_D:], zrows], axis=0)
    pad = E_PAD - E
    rows_p = jnp.concatenate(
        [adj_rows.astype(jnp.int32), jnp.zeros((pad,), jnp.int32)])
    cols_p = jnp.concatenate(
        [adj_cols.astype(jnp.int32), jnp.zeros((pad,), jnp.int32)])
    vals_p = jnp.concatenate([adj_vals, jnp.zeros((pad,), jnp.float32)])
    cols_r = cols_p.reshape(IDX_ROWS, 128)
    rows_r = rows_p.reshape(IDX_ROWS, 128)
    vals_r = vals_p.reshape(IDX_ROWS, 128)

    out, _, _, _ = _gcl(ego_h, cols_r, rows_r, vals_r)
    return out[: N_NODES // 2], out[N_NODES // 2: N_NODES]


# speedup vs baseline: 10.6855x; 1.0662x over previous
import jax
import jax.numpy as jnp
from jax import lax
from jax.experimental import pallas as pl
from jax.experimental.pallas import tpu as pltpu
from jax.experimental.pallas import tpu_sc as plsc

N_NODES = 50000
N_TBL = 50048
HALF_D = 32
E = 800000
N_LAYERS = 3
NC, NS = 2, 16
NBUF = 6

T_STREAMS = 396
E_TILE = T_STREAMS * 128
E_PAD = E_TILE * NS
IDX_ROWS = E_PAD // 128
ROWS_PER_TILE = N_TBL // NS
N_GRP = T_STREAMS // NBUF


def _body(ego0, cols, rows, vals, out, t1, t2, t3,
          acc, ebuf, gbuf, sem_e, sem_g, sem_s, sem_f):
    c = lax.axis_index("c")
    s = lax.axis_index("s")
    half_base = c * N_TBL + s * ROWS_PER_TILE
    ebase = s * T_STREAMS
    zeros16 = jnp.zeros((16,), jnp.float32)

    def run_layer(src_tbl, dst_tbl):
        @pl.loop(0, 128)
        def _zb(r):
            gbuf[0, r, 0:16] = zeros16
            gbuf[0, r, 16:32] = zeros16

        for q in range(24):
            pltpu.sync_copy(
                gbuf.at[0],
                acc.at[pl.ds(s * ROWS_PER_TILE + q * 128, 128)])
        pltpu.sync_copy(
            gbuf.at[0].at[pl.ds(0, 56)],
            acc.at[pl.ds(s * ROWS_PER_TILE + 3072, 56)])
        plsc.subcore_barrier()

        def stage(kk, j):
            pltpu.async_copy(cols.at[ebase + kk], ebuf.at[j].at[0],
                             sem_e.at[j])
            pltpu.async_copy(rows.at[ebase + kk], ebuf.at[j].at[1],
                             sem_e.at[j])
            pltpu.async_copy(vals.at[ebase + kk], ebuf.at[j].at[2],
                             sem_e.at[j])

        def wait_e(j):
            pltpu.make_async_copy(cols.at[pl.ds(0, 3)], ebuf.at[j],
                                  sem_e.at[j]).wait()
            for i in range(8):
                sl = pl.ds(i * 16, 16)
                ebuf[j, 0, sl] = ebuf[j, 0, sl] + c * N_TBL

        def gather(j):
            pltpu.async_copy(src_tbl.at[ebuf.at[j].at[0]],
                             gbuf.at[j], sem_g.at[j])

        def wait_g(j):
            pltpu.make_async_copy(src_tbl.at[ebuf.at[j].at[0]],
                                  gbuf.at[j], sem_g.at[j]).wait()

        def scatter(j):
            pltpu.async_copy(gbuf.at[j], acc.at[ebuf.at[j].at[1]],
                             sem_s.at[j], add=True)

        def wait_s(j):
            pltpu.make_async_copy(gbuf.at[j], acc.at[ebuf.at[j].at[1]],
                                  sem_s.at[j]).wait()

        def scale(j):
            @pl.loop(0, 8)
            def _scale(i):
                vv = plsc.bitcast(
                    ebuf[j, 2, pl.ds(i * 16, 16)], jnp.float32)
                for q in range(16):
                    e = i * 16 + q
                    v = vv[q]
                    gbuf[j, e, 0:16] = gbuf[j, e, 0:16] * v
                    gbuf[j, e, 16:32] = gbuf[j, e, 16:32] * v

        def step(kk, j):
            wait_g(j)
            scale(j)
            scatter(j)

            @pl.when(kk >= 2)
            def _ws():
                wait_s((j + 4) % NBUF)

            @pl.when(kk + 4 <= T_STREAMS - 1)
            def _st():
                stage(kk + 4, (j + 4) % NBUF)

            @pl.when(kk + 3 <= T_STREAMS - 1)
            def _ga():
                wait_e((j + 3) % NBUF)
                gather((j + 3) % NBUF)

        for j in range(4):
            stage(j, j)
        for j in range(3):
            wait_e(j)
            gather(j)

        @pl.loop(0, N_GRP)
        def _grp(g):
            for j in range(NBUF):
                step(NBUF * g + j, j)

        for j in (4, 5):
            wait_s(j)

        plsc.subcore_barrier()
        pltpu.sync_copy(acc.at[pl.ds(s * ROWS_PER_TILE, ROWS_PER_TILE)],
                        dst_tbl.at[pl.ds(half_base, ROWS_PER_TILE)])

    srcs = [ego0, t1, t2]
    dsts = [t1, t2, t3]
    for l in range(N_LAYERS):
        run_layer(srcs[l], dsts[l])

    def fin_chunk(base, nrows):
        tbls = (ego0, t1, t2, t3)
        for i, tbl in enumerate(tbls):
            pltpu.async_copy(tbl.at[pl.ds(base, nrows)],
                             gbuf.at[i].at[pl.ds(0, nrows)], sem_f.at[i])
        for i, tbl in enumerate(tbls):
            pltpu.make_async_copy(tbl.at[pl.ds(base, nrows)],
                                  gbuf.at[i].at[pl.ds(0, nrows)],
                                  sem_f.at[i]).wait()

        @pl.loop(0, nrows)
        def _avg(r):
            for h in (0, 16):
                gbuf[4, r, h:h + 16] = (
                    (gbuf[0, r, h:h + 16] + gbuf[1, r, h:h + 16])
                    + (gbuf[2, r, h:h + 16] + gbuf[3, r, h:h + 16])) * 0.25

        pltpu.sync_copy(
            gbuf.at[4].at[pl.ds(0, nrows)],
            out.at[pl.ds(base - c * N_TBL, nrows),
                   pl.ds(c * HALF_D, HALF_D)])

    @pl.loop(0, 24)
    def _fin(t):
        fin_chunk(half_base + t * 128, 128)

    fin_chunk(half_base + 3072, 56)


_mesh = plsc.VectorSubcoreMesh(
    core_axis_name="c", subcore_axis_name="s", num_cores=NC, num_subcores=NS)

_tbl = jax.ShapeDtypeStruct((2 * N_TBL, HALF_D), jnp.float32)
_out_t = jax.ShapeDtypeStruct((N_TBL, 2 * HALF_D), jnp.float32)

_gcl = pl.kernel(
    _body,
    out_type=(_out_t, _tbl, _tbl, _tbl),
    mesh=_mesh,
    compiler_params=pltpu.CompilerParams(
        use_tc_tiling_on_sc=False, needs_layout_passes=False),
    scratch_types=[
        pltpu.VMEM_SHARED((N_TBL, HALF_D), jnp.float32),
        pltpu.VMEM((NBUF, 3, 128), jnp.int32),
        pltpu.VMEM((NBUF, 128, HALF_D), jnp.float32),
        pltpu.SemaphoreType.DMA((NBUF,)),
        pltpu.SemaphoreType.DMA((NBUF,)),
        pltpu.SemaphoreType.DMA((NBUF,)),
        pltpu.SemaphoreType.DMA((4,)),
    ],
)


@jax.jit
def kernel(user_emb, item_emb, adj_rows, adj_cols, adj_vals):
    ego = jnp.concatenate([user_emb, item_emb], axis=0)
    zrows = jnp.zeros((N_TBL - N_NODES, HALF_D), jnp.float32)
    ego_h = jnp.concatenate(
        [ego[:, :HALF_D], zrows, ego[:, HALF_D:], zrows], axis=0)
    pad = E_PAD - E
    rows_p = jnp.concatenate(
        [adj_rows.astype(jnp.int32), jnp.zeros((pad,), jnp.int32)])
    cols_p = jnp.concatenate(
        [adj_cols.astype(jnp.int32), jnp.zeros((pad,), jnp.int32)])
    vals_p = jnp.concatenate([adj_vals, jnp.zeros((pad,), jnp.float32)])
    cols_r = cols_p.reshape(IDX_ROWS, 128)
    rows_r = rows_p.reshape(IDX_ROWS, 128)
    vals_r = lax.bitcast_convert_type(vals_p, jnp.int32).reshape(
        IDX_ROWS, 128)

    out, _, _, _ = _gcl(ego_h, cols_r, rows_r, vals_r)
    return out[: N_NODES // 2], out[N_NODES // 2: N_NODES]


# speedup vs baseline: 10.8247x; 1.0130x over previous
import jax
import jax.numpy as jnp
from jax import lax
from jax.experimental import pallas as pl
from jax.experimental.pallas import tpu as pltpu
from jax.experimental.pallas import tpu_sc as plsc

N_NODES = 50000
N_TBL = 50048
HALF_D = 32
E = 800000
N_LAYERS = 3
NC, NS = 2, 16
NBUF = 6

T_STREAMS = 396
E_TILE = T_STREAMS * 128
E_PAD = E_TILE * NS
IDX_ROWS = E_PAD // 128
ROWS_PER_TILE = N_TBL // NS
N_GRP = T_STREAMS // NBUF


def _body(ego0, cols, rows, vals, out, t1, t2,
          acc, ebuf, gbuf, sem_e, sem_g, sem_s, sem_f):
    c = lax.axis_index("c")
    s = lax.axis_index("s")
    half_base = c * N_TBL + s * ROWS_PER_TILE
    ebase = s * T_STREAMS
    zeros16 = jnp.zeros((16,), jnp.float32)

    def run_layer(src_tbl, dst_tbl):
        @pl.loop(0, 128)
        def _zb(r):
            gbuf[0, r, 0:16] = zeros16
            gbuf[0, r, 16:32] = zeros16

        for q in range(24):
            pltpu.async_copy(
                gbuf.at[0],
                acc.at[pl.ds(s * ROWS_PER_TILE + q * 128, 128)],
                sem_f.at[q % 4])
        pltpu.sync_copy(
            gbuf.at[0].at[pl.ds(0, 56)],
            acc.at[pl.ds(s * ROWS_PER_TILE + 3072, 56)])
        for q in range(24):
            pltpu.make_async_copy(
                gbuf.at[0],
                acc.at[pl.ds(s * ROWS_PER_TILE, 128)],
                sem_f.at[q % 4]).wait()
        plsc.subcore_barrier()

        def stage(kk, j):
            pltpu.async_copy(cols.at[ebase + kk], ebuf.at[j].at[0],
                             sem_e.at[j])
            pltpu.async_copy(rows.at[ebase + kk], ebuf.at[j].at[1],
                             sem_e.at[j])
            pltpu.async_copy(vals.at[ebase + kk], ebuf.at[j].at[2],
                             sem_e.at[j])

        def wait_e(j):
            pltpu.make_async_copy(cols.at[pl.ds(0, 3)], ebuf.at[j],
                                  sem_e.at[j]).wait()
            for i in range(8):
                sl = pl.ds(i * 16, 16)
                ebuf[j, 0, sl] = ebuf[j, 0, sl] + c * N_TBL

        def gather(j):
            pltpu.async_copy(src_tbl.at[ebuf.at[j].at[0]],
                             gbuf.at[j], sem_g.at[j])

        def wait_g(j):
            pltpu.make_async_copy(src_tbl.at[ebuf.at[j].at[0]],
                                  gbuf.at[j], sem_g.at[j]).wait()

        def scatter(j):
            pltpu.async_copy(gbuf.at[j], acc.at[ebuf.at[j].at[1]],
                             sem_s.at[j], add=True)

        def wait_s(j):
            pltpu.make_async_copy(gbuf.at[j], acc.at[ebuf.at[j].at[1]],
                                  sem_s.at[j]).wait()

        def scale(j):
            @pl.loop(0, 8)
            def _scale(i):
                vv = plsc.bitcast(
                    ebuf[j, 2, pl.ds(i * 16, 16)], jnp.float32)
                for q in range(16):
                    e = i * 16 + q
                    v = vv[q]
                    gbuf[j, e, 0:16] = gbuf[j, e, 0:16] * v
                    gbuf[j, e, 16:32] = gbuf[j, e, 16:32] * v

        def step(kk, j):
            wait_g(j)
            scale(j)
            scatter(j)

            @pl.when(kk >= 2)
            def _ws():
                wait_s((j + 4) % NBUF)

            @pl.when(kk + 4 <= T_STREAMS - 1)
            def _st():
                stage(kk + 4, (j + 4) % NBUF)

            @pl.when(kk + 3 <= T_STREAMS - 1)
            def _ga():
                wait_e((j + 3) % NBUF)
                gather((j + 3) % NBUF)

        for j in range(4):
            stage(j, j)
        for j in range(3):
            wait_e(j)
            gather(j)

        @pl.loop(0, N_GRP)
        def _grp(g):
            for j in range(NBUF):
                step(NBUF * g + j, j)

        for j in (4, 5):
            wait_s(j)

        plsc.subcore_barrier()
        if dst_tbl is not None:
            pltpu.sync_copy(
                acc.at[pl.ds(s * ROWS_PER_TILE, ROWS_PER_TILE)],
                dst_tbl.at[pl.ds(half_base, ROWS_PER_TILE)])

    srcs = [ego0, t1, t2]
    dsts = [t1, t2, None]
    for l in range(N_LAYERS):
        run_layer(srcs[l], dsts[l])

    def fin_chunk(q, nrows):
        base = s * ROWS_PER_TILE + q * 128
        tbase = half_base + q * 128
        for i, tbl in enumerate((ego0, t1, t2)):
            pltpu.async_copy(tbl.at[pl.ds(tbase, nrows)],
                             gbuf.at[i].at[pl.ds(0, nrows)], sem_f.at[i])
        pltpu.async_copy(acc.at[pl.ds(base, nrows)],
                         gbuf.at[3].at[pl.ds(0, nrows)], sem_f.at[3])
        for i, tbl in enumerate((ego0, t1, t2)):
            pltpu.make_async_copy(tbl.at[pl.ds(tbase, nrows)],
                                  gbuf.at[i].at[pl.ds(0, nrows)],
                                  sem_f.at[i]).wait()
        pltpu.make_async_copy(acc.at[pl.ds(base, nrows)],
                              gbuf.at[3].at[pl.ds(0, nrows)],
                              sem_f.at[3]).wait()

        @pl.loop(0, nrows)
        def _avg(r):
            for h in (0, 16):
                gbuf[4, r, h:h + 16] = (
                    (gbuf[0, r, h:h + 16] + gbuf[1, r, h:h + 16])
                    + (gbuf[2, r, h:h + 16] + gbuf[3, r, h:h + 16])) * 0.25

        pltpu.sync_copy(
            gbuf.at[4].at[pl.ds(0, nrows)],
            out.at[pl.ds(base, nrows), pl.ds(c * HALF_D, HALF_D)])

    @pl.loop(0, 24)
    def _fin(t):
        fin_chunk(t, 128)

    fin_chunk(24, 56)


_mesh = plsc.VectorSubcoreMesh(
    core_axis_name="c", subcore_axis_name="s", num_cores=NC, num_subcores=NS)

_tbl = jax.ShapeDtypeStruct((2 * N_TBL, HALF_D), jnp.float32)
_out_t = jax.ShapeDtypeStruct((N_TBL, 2 * HALF_D), jnp.float32)

_gcl = pl.kernel(
    _body,
    out_type=(_out_t, _tbl, _tbl),
    mesh=_mesh,
    compiler_params=pltpu.CompilerParams(
        use_tc_tiling_on_sc=False, needs_layout_passes=False),
    scratch_types=[
        pltpu.VMEM_SHARED((N_TBL, HALF_D), jnp.float32),
        pltpu.VMEM((NBUF, 3, 128), jnp.int32),
        pltpu.VMEM((NBUF, 128, HALF_D), jnp.float32),
        pltpu.SemaphoreType.DMA((NBUF,)),
        pltpu.SemaphoreType.DMA((NBUF,)),
        pltpu.SemaphoreType.DMA((NBUF,)),
        pltpu.SemaphoreType.DMA((4,)),
    ],
)


@jax.jit
def kernel(user_emb, item_emb, adj_rows, adj_cols, adj_vals):
    ego = jnp.concatenate([user_emb, item_emb], axis=0)
    zrows = jnp.zeros((N_TBL - N_NODES, HALF_D), jnp.float32)
    ego_h = jnp.concatenate(
        [ego[:, :HALF_D], zrows, ego[:, HALF_D:], zrows], axis=0)
    pad = E_PAD - E
    rows_p = jnp.concatenate(
        [adj_rows.astype(jnp.int32), jnp.zeros((pad,), jnp.int32)])
    cols_p = jnp.concatenate(
        [adj_cols.astype(jnp.int32), jnp.zeros((pad,), jnp.int32)])
    vals_p = jnp.concatenate([adj_vals, jnp.zeros((pad,), jnp.float32)])
    cols_r = cols_p.reshape(IDX_ROWS, 128)
    rows_r = rows_p.reshape(IDX_ROWS, 128)
    vals_r = lax.bitcast_convert_type(vals_p, jnp.int32).reshape(
        IDX_ROWS, 128)

    out, _, _ = _gcl(ego_h, cols_r, rows_r, vals_r)
    return out[: N_NODES // 2], out[N_NODES // 2: N_NODES]


# speedup vs baseline: 10.8399x; 1.0014x over previous
import jax
import jax.numpy as jnp
from jax import lax
from jax.experimental import pallas as pl
from jax.experimental.pallas import tpu as pltpu
from jax.experimental.pallas import tpu_sc as plsc

N_NODES = 50000
N_TBL = 50048
HALF_D = 32
E = 800000
N_LAYERS = 3
NC, NS = 2, 16
NBUF = 6

T_STREAMS = 396
E_TILE = T_STREAMS * 128
E_PAD = E_TILE * NS
IDX_ROWS = E_PAD // 128
ROWS_PER_TILE = N_TBL // NS
N_GRP = T_STREAMS // NBUF


def _body(ego0, cols, rows, vals, out, t1, t2,
          acc, ebuf, gbuf, sem_e, sem_g, sem_s, sem_f):
    c = lax.axis_index("c")
    s = lax.axis_index("s")
    half_base = c * N_TBL + s * ROWS_PER_TILE
    ebase = s * T_STREAMS
    zeros16 = jnp.zeros((16,), jnp.float32)

    def run_layer(src_tbl, dst_tbl):
        @pl.loop(0, 128)
        def _zb(r):
            gbuf[0, r, 0:16] = zeros16
            gbuf[0, r, 16:32] = zeros16

        for q in range(24):
            pltpu.async_copy(
                gbuf.at[0],
                acc.at[pl.ds(s * ROWS_PER_TILE + q * 128, 128)],
                sem_f.at[q % 4])
        pltpu.sync_copy(
            gbuf.at[0].at[pl.ds(0, 56)],
            acc.at[pl.ds(s * ROWS_PER_TILE + 3072, 56)])
        for q in range(24):
            pltpu.make_async_copy(
                gbuf.at[0],
                acc.at[pl.ds(s * ROWS_PER_TILE, 128)],
                sem_f.at[q % 4]).wait()
        plsc.subcore_barrier()

        def stage(kk, j):
            pltpu.async_copy(cols.at[ebase + kk], ebuf.at[j].at[0],
                             sem_e.at[j])
            pltpu.async_copy(rows.at[ebase + kk], ebuf.at[j].at[1],
                             sem_e.at[j])
            pltpu.async_copy(vals.at[ebase + kk], ebuf.at[j].at[2],
                             sem_e.at[j])

        def wait_e(j):
            pltpu.make_async_copy(cols.at[pl.ds(0, 3)], ebuf.at[j],
                                  sem_e.at[j]).wait()
            for i in range(8):
                sl = pl.ds(i * 16, 16)
                ebuf[j, 0, sl] = ebuf[j, 0, sl] + c * N_TBL

        def gather(j):
            for h in (0, 64):
                pltpu.async_copy(
                    src_tbl.at[ebuf.at[j].at[0].at[pl.ds(h, 64)]],
                    gbuf.at[j].at[pl.ds(h, 64)], sem_g.at[j])

        def wait_g(j):
            for h in (0, 64):
                pltpu.make_async_copy(
                    src_tbl.at[ebuf.at[j].at[0].at[pl.ds(h, 64)]],
                    gbuf.at[j].at[pl.ds(h, 64)], sem_g.at[j]).wait()

        def scatter(j):
            pltpu.async_copy(gbuf.at[j], acc.at[ebuf.at[j].at[1]],
                             sem_s.at[j], add=True)

        def wait_s(j):
            pltpu.make_async_copy(gbuf.at[j], acc.at[ebuf.at[j].at[1]],
                                  sem_s.at[j]).wait()

        def scale(j):
            @pl.loop(0, 8)
            def _scale(i):
                vv = plsc.bitcast(
                    ebuf[j, 2, pl.ds(i * 16, 16)], jnp.float32)
                for q in range(16):
                    e = i * 16 + q
                    v = vv[q]
                    gbuf[j, e, 0:16] = gbuf[j, e, 0:16] * v
                    gbuf[j, e, 16:32] = gbuf[j, e, 16:32] * v

        def step(kk, j):
            wait_g(j)
            scale(j)
            scatter(j)

            @pl.when(kk >= 2)
            def _ws():
                wait_s((j + 4) % NBUF)

            @pl.when(kk + 4 <= T_STREAMS - 1)
            def _st():
                stage(kk + 4, (j + 4) % NBUF)

            @pl.when(kk + 3 <= T_STREAMS - 1)
            def _ga():
                wait_e((j + 3) % NBUF)
                gather((j + 3) % NBUF)

        for j in range(4):
            stage(j, j)
        for j in range(3):
            wait_e(j)
            gather(j)

        @pl.loop(0, N_GRP)
        def _grp(g):
            for j in range(NBUF):
                step(NBUF * g + j, j)

        for j in (4, 5):
            wait_s(j)

        plsc.subcore_barrier()
        if dst_tbl is not None:
            pltpu.sync_copy(
                acc.at[pl.ds(s * ROWS_PER_TILE, ROWS_PER_TILE)],
                dst_tbl.at[pl.ds(half_base, ROWS_PER_TILE)])

    srcs = [ego0, t1, t2]
    dsts = [t1, t2, None]
    for l in range(N_LAYERS):
        run_layer(srcs[l], dsts[l])

    def fin_chunk(q, nrows):
        base = s * ROWS_PER_TILE + q * 128
        tbase = half_base + q * 128
        for i, tbl in enumerate((ego0, t1, t2)):
            pltpu.async_copy(tbl.at[pl.ds(tbase, nrows)],
                             gbuf.at[i].at[pl.ds(0, nrows)], sem_f.at[i])
        pltpu.async_copy(acc.at[pl.ds(base, nrows)],
                         gbuf.at[3].at[pl.ds(0, nrows)], sem_f.at[3])
        for i, tbl in enumerate((ego0, t1, t2)):
            pltpu.make_async_copy(tbl.at[pl.ds(tbase, nrows)],
                                  gbuf.at[i].at[pl.ds(0, nrows)],
                                  sem_f.at[i]).wait()
        pltpu.make_async_copy(acc.at[pl.ds(base, nrows)],
                              gbuf.at[3].at[pl.ds(0, nrows)],
                              sem_f.at[3]).wait()

        @pl.loop(0, nrows)
        def _avg(r):
            for h in (0, 16):
                gbuf[4, r, h:h + 16] = (
                    (gbuf[0, r, h:h + 16] + gbuf[1, r, h:h + 16])
                    + (gbuf[2, r, h:h + 16] + gbuf[3, r, h:h + 16])) * 0.25

        pltpu.sync_copy(
            gbuf.at[4].at[pl.ds(0, nrows)],
            out.at[pl.ds(base, nrows), pl.ds(c * HALF_D, HALF_D)])

    @pl.loop(0, 24)
    def _fin(t):
        fin_chunk(t, 128)

    fin_chunk(24, 56)


_mesh = plsc.VectorSubcoreMesh(
    core_axis_name="c", subcore_axis_name="s", num_cores=NC, num_subcores=NS)

_tbl = jax.ShapeDtypeStruct((2 * N_TBL, HALF_D), jnp.float32)
_out_t = jax.ShapeDtypeStruct((N_TBL, 2 * HALF_D), jnp.float32)

_gcl = pl.kernel(
    _body,
    out_type=(_out_t, _tbl, _tbl),
    mesh=_mesh,
    compiler_params=pltpu.CompilerParams(
        use_tc_tiling_on_sc=False, needs_layout_passes=False),
    scratch_types=[
        pltpu.VMEM_SHARED((N_TBL, HALF_D), jnp.float32),
        pltpu.VMEM((NBUF, 3, 128), jnp.int32),
        pltpu.VMEM((NBUF, 128, HALF_D), jnp.float32),
        pltpu.SemaphoreType.DMA((NBUF,)),
        pltpu.SemaphoreType.DMA((NBUF,)),
        pltpu.SemaphoreType.DMA((NBUF,)),
        pltpu.SemaphoreType.DMA((4,)),
    ],
)


@jax.jit
def kernel(user_emb, item_emb, adj_rows, adj_cols, adj_vals):
    ego = jnp.concatenate([user_emb, item_emb], axis=0)
    zrows = jnp.zeros((N_TBL - N_NODES, HALF_D), jnp.float32)
    ego_h = jnp.concatenate(
        [ego[:, :HALF_D], zrows, ego[:, HALF_D:], zrows], axis=0)
    pad = E_PAD - E
    rows_p = jnp.concatenate(
        [adj_rows.astype(jnp.int32), jnp.zeros((pad,), jnp.int32)])
    cols_p = jnp.concatenate(
        [adj_cols.astype(jnp.int32), jnp.zeros((pad,), jnp.int32)])
    vals_p = jnp.concatenate([adj_vals, jnp.zeros((pad,), jnp.float32)])
    cols_r = cols_p.reshape(IDX_ROWS, 128)
    rows_r = rows_p.reshape(IDX_ROWS, 128)
    vals_r = lax.bitcast_convert_type(vals_p, jnp.int32).reshape(
        IDX_ROWS, 128)

    out, _, _ = _gcl(ego_h, cols_r, rows_r, vals_r)
    return out[: N_NODES // 2], out[N_NODES // 2: N_NODES]
